# baseline (speedup 1.0000x reference)
import jax
import jax.numpy as jnp
from jax.experimental import pallas as pl
from jax.experimental.pallas import tpu as pltpu

N = 10000
NP = 10240
PW = 1024
NPAN = NP // PW
N_USER = 6000
D = 64
BM = 400
NBLK = N // BM
BS = 1000
NSW = N // BS
BP = 2000
UBLKP = N_USER // BP
ALPHA = 0.25

_PATCH_OFFS = (0, 10, 18, 24, 28)
_NPATCH = 30


def _patch_ij(tp):
    big_i = (jnp.where(tp >= _PATCH_OFFS[1], 1, 0)
             + jnp.where(tp >= _PATCH_OFFS[2], 1, 0)
             + jnp.where(tp >= _PATCH_OFFS[3], 1, 0)
             + jnp.where(tp >= _PATCH_OFFS[4], 1, 0))
    off = (jnp.where(big_i == 1, _PATCH_OFFS[1], 0)
           + jnp.where(big_i == 2, _PATCH_OFFS[2], 0)
           + jnp.where(big_i == 3, _PATCH_OFFS[3], 0)
           + jnp.where(big_i == 4, _PATCH_OFFS[4], 0))
    j = 2 * big_i + (tp - off)
    return big_i, j


def _pass1_kernel(a_ref, u_in, v_in, r1_ref, a16_ref, x_ref):
    t = pl.program_id(0)
    i = jnp.maximum(t - 1, 0)
    rows = pl.ds(i * BM, BM)

    @pl.when(t == 0)
    def _assemble_x():
        x_ref[:N_USER, :] = u_in[...].astype(jnp.bfloat16)
        x_ref[N_USER:, :] = v_in[...].astype(jnp.bfloat16)

    @pl.when(t > 0)
    def _layer0():
        a16 = a_ref[...].astype(jnp.bfloat16)
        a16_ref[...] = jnp.concatenate(
            [a16, jnp.zeros((BM, NP - N), jnp.bfloat16)], axis=1)
        r1_ref[...] = (ALPHA * jnp.dot(
            a16, x_ref[...], preferred_element_type=jnp.float32
        ) + ALPHA * x_ref[rows, :].astype(jnp.float32)).astype(jnp.bfloat16)


def _sweep_kernel(a16_ref, r1_in, u_in, v_in, r2_ref, op_ref, r1p_ref,
                  r2p_ref, acc_ref):
    t = pl.program_id(0)
    i = jnp.maximum(t - 1, 0)
    rows = pl.ds(i * BS, BS)

    @pl.when(t == 0)
    def _setup():
        r1p_ref[:N, :] = r1_in[...]
        r1p_ref[N:, :] = jnp.zeros((NP - N, D), jnp.bfloat16)
        r2p_ref[N:, :] = jnp.zeros((NP - N, D), jnp.bfloat16)

    @pl.when(t > 0)
    def _sweep():
        bias = ALPHA * jnp.where(
            (i * BS) < N_USER,
            u_in[pl.ds(jnp.minimum(i * BS, N_USER - BS), BS), :],
            v_in[pl.ds(jnp.maximum(i * BS - N_USER, 0), BS), :])
        r2c = (jnp.dot(a16_ref[...], r1p_ref[...],
                       preferred_element_type=jnp.float32)
               + bias).astype(jnp.bfloat16)
        r2_ref[...] = r2c
        r2p_ref[rows, :] = r2c
        ndone = i - jnp.remainder(i, 2)
        acc_ref[...] = jnp.zeros((BS, D), jnp.float32)
        for j in range(NPAN - 2):
            @pl.when(j < ndone)
            def _panel(j=j):
                acc_ref[...] += jnp.dot(
                    a16_ref[:, j * PW:(j + 1) * PW],
                    r2p_ref[pl.ds(j * PW, PW), :],
                    preferred_element_type=jnp.float32)
        op_ref[...] = acc_ref[...]


def _patch_kernel(a16_ref, r2_in, op_in, u_in, v_in, u_ref, v_ref, r2p_ref,
                  acc_ref):
    t = pl.program_id(0)
    tp = jnp.maximum(t - 1, 0)
    big_i, j = _patch_ij(tp)

    @pl.when(t == 0)
    def _setup():
        r2p_ref[:N, :] = r2_in[...]
        r2p_ref[N:, :] = jnp.zeros((NP - N, D), jnp.bfloat16)

    @pl.when(t > 0)
    def _patch():
        prod = jnp.dot(a16_ref[...], r2p_ref[pl.ds(j * PW, PW), :],
                       preferred_element_type=jnp.float32)

        @pl.when(j == 2 * big_i)
        def _first():
            acc_ref[...] = op_in[...] + prod

        @pl.when(j != 2 * big_i)
        def _rest():
            acc_ref[...] += prod

        @pl.when(j == NPAN - 1)
        def _emit():
            res = acc_ref[...] + ALPHA * jnp.where(
                (big_i * BP) < N_USER,
                u_in[pl.ds(jnp.minimum(big_i * BP, N_USER - BP), BP), :],
                v_in[pl.ds(jnp.maximum(big_i * BP - N_USER, 0), BP), :])

            @pl.when(big_i < UBLKP)
            def _u():
                u_ref[...] = res

            @pl.when(big_i >= UBLKP)
            def _v():
                v_ref[...] = res


def kernel(user_embedding, item_embedding, norm_adj):
    r1, a16 = pl.pallas_call(
        _pass1_kernel,
        grid=(1 + NBLK,),
        in_specs=[
            pl.BlockSpec((BM, N), lambda t: (jnp.maximum(t - 1, 0), 0)),
            pl.BlockSpec((N_USER, D), lambda t: (0, 0)),
            pl.BlockSpec((N - N_USER, D), lambda t: (0, 0)),
        ],
        out_specs=[
            pl.BlockSpec((BM, D), lambda t: (jnp.maximum(t - 1, 0), 0)),
            pl.BlockSpec((BM, NP), lambda t: (jnp.maximum(t - 1, 0), 0)),
        ],
        out_shape=[
            jax.ShapeDtypeStruct((N, D), jnp.bfloat16),
            jax.ShapeDtypeStruct((N, NP), jnp.bfloat16),
        ],
        scratch_shapes=[pltpu.VMEM((N, D), jnp.bfloat16)],
        compiler_params=pltpu.CompilerParams(
            dimension_semantics=("arbitrary",)),
    )(norm_adj, user_embedding, item_embedding)

    r2, opart = pl.pallas_call(
        _sweep_kernel,
        grid=(1 + NSW,),
        in_specs=[
            pl.BlockSpec((BS, NP), lambda t: (jnp.maximum(t - 1, 0), 0)),
            pl.BlockSpec((N, D), lambda t: (0, 0)),
            pl.BlockSpec((N_USER, D), lambda t: (0, 0)),
            pl.BlockSpec((N - N_USER, D), lambda t: (0, 0)),
        ],
        out_specs=[
            pl.BlockSpec((BS, D), lambda t: (jnp.maximum(t - 1, 0), 0)),
            pl.BlockSpec((BS, D), lambda t: (jnp.maximum(t - 1, 0), 0)),
        ],
        out_shape=[
            jax.ShapeDtypeStruct((N, D), jnp.bfloat16),
            jax.ShapeDtypeStruct((N, D), jnp.float32),
        ],
        scratch_shapes=[
            pltpu.VMEM((NP, D), jnp.bfloat16),
            pltpu.VMEM((NP, D), jnp.bfloat16),
            pltpu.VMEM((BS, D), jnp.float32),
        ],
        compiler_params=pltpu.CompilerParams(
            dimension_semantics=("arbitrary",)),
    )(a16, r1, user_embedding, item_embedding)

    def _a16p_map(t):
        big_i, j = _patch_ij(jnp.maximum(t - 1, 0))
        return (big_i, j)

    def _u_map(t):
        big_i, _ = _patch_ij(jnp.maximum(t - 1, 0))
        return (jnp.minimum(big_i, UBLKP - 1), 0)

    def _v_map(t):
        big_i, _ = _patch_ij(jnp.maximum(t - 1, 0))
        return (jnp.maximum(big_i - UBLKP, 0), 0)

    def _op_map(t):
        big_i, _ = _patch_ij(jnp.maximum(t - 1, 0))
        return (big_i, 0)

    u_out, v_out = pl.pallas_call(
        _patch_kernel,
        grid=(1 + _NPATCH,),
        in_specs=[
            pl.BlockSpec((BP, PW), _a16p_map),
            pl.BlockSpec((N, D), lambda t: (0, 0)),
            pl.BlockSpec((BP, D), _op_map),
            pl.BlockSpec((N_USER, D), lambda t: (0, 0)),
            pl.BlockSpec((N - N_USER, D), lambda t: (0, 0)),
        ],
        out_specs=[
            pl.BlockSpec((BP, D), _u_map),
            pl.BlockSpec((BP, D), _v_map),
        ],
        out_shape=[
            jax.ShapeDtypeStruct((N_USER, D), jnp.float32),
            jax.ShapeDtypeStruct((N - N_USER, D), jnp.float32),
        ],
        scratch_shapes=[
            pltpu.VMEM((NP, D), jnp.bfloat16),
            pltpu.VMEM((BP, D), jnp.float32),
        ],
        compiler_params=pltpu.CompilerParams(
            dimension_semantics=("arbitrary",)),
    )(a16, r2, opart, user_embedding, item_embedding)
    return (u_out, v_out)


# speedup vs baseline: 1.0897x; 1.0897x over previous
import jax
import jax.numpy as jnp
from jax.experimental import pallas as pl
from jax.experimental.pallas import tpu as pltpu

N = 10000
N_USER = 6000
D = 64
BM = 400
NBLK = N // BM
BM2 = 1000
NBLK2 = N // BM2
UBLK2 = N_USER // BM2
ALPHA = 0.25


def _pass1_kernel(a_ref, u_in, v_in, r1_ref, a16_ref, x_ref):
    t = pl.program_id(0)
    i = jnp.maximum(t - 1, 0)
    rows = pl.ds(i * BM, BM)

    @pl.when(t == 0)
    def _assemble_x():
        x_ref[:N_USER, :] = u_in[...].astype(jnp.bfloat16)
        x_ref[N_USER:, :] = v_in[...].astype(jnp.bfloat16)

    @pl.when(t > 0)
    def _layer0():
        a16 = a_ref[...].astype(jnp.bfloat16)
        a16_ref[...] = a16
        r1_ref[...] = (ALPHA * jnp.dot(
            a16, x_ref[...], preferred_element_type=jnp.float32
        ) + ALPHA * x_ref[rows, :].astype(jnp.float32)).astype(jnp.bfloat16)


def _pass2_kernel(a16_ref, r1_in, u_in, v_in, u_ref, v_ref, r2_ref):
    t = pl.program_id(0)
    l = t // NBLK2
    i = t % NBLK2
    rows = pl.ds(i * BM2, BM2)
    urows = pl.ds(i * BM2, BM2)
    vrows = pl.ds(i * BM2 - N_USER, BM2)

    @pl.when(jnp.logical_and(l == 0, i < UBLK2))
    def _layer1_user():
        r2_ref[rows, :] = (jnp.dot(a16_ref[...], r1_in[...],
                                   preferred_element_type=jnp.float32)
                           + ALPHA * u_in[urows, :]).astype(jnp.bfloat16)

    @pl.when(jnp.logical_and(l == 0, i >= UBLK2))
    def _layer1_item():
        r2_ref[rows, :] = (jnp.dot(a16_ref[...], r1_in[...],
                                   preferred_element_type=jnp.float32)
                           + ALPHA * v_in[vrows, :]).astype(jnp.bfloat16)

    @pl.when(jnp.logical_and(l == 1, i < UBLK2))
    def _layer2_user():
        u_ref[...] = jnp.dot(a16_ref[...], r2_ref[...],
                             preferred_element_type=jnp.float32
                             ) + ALPHA * u_in[urows, :]

    @pl.when(jnp.logical_and(l == 1, i >= UBLK2))
    def _layer2_item():
        v_ref[...] = jnp.dot(a16_ref[...], r2_ref[...],
                             preferred_element_type=jnp.float32
                             ) + ALPHA * v_in[vrows, :]


def kernel(user_embedding, item_embedding, norm_adj):
    r1, a16 = pl.pallas_call(
        _pass1_kernel,
        grid=(1 + NBLK,),
        in_specs=[
            pl.BlockSpec((BM, N), lambda t: (jnp.maximum(t - 1, 0), 0)),
            pl.BlockSpec((N_USER, D), lambda t: (0, 0)),
            pl.BlockSpec((N - N_USER, D), lambda t: (0, 0)),
        ],
        out_specs=[
            pl.BlockSpec((BM, D), lambda t: (jnp.maximum(t - 1, 0), 0)),
            pl.BlockSpec((BM, N), lambda t: (jnp.maximum(t - 1, 0), 0)),
        ],
        out_shape=[
            jax.ShapeDtypeStruct((N, D), jnp.bfloat16),
            jax.ShapeDtypeStruct((N, N), jnp.bfloat16),
        ],
        scratch_shapes=[pltpu.VMEM((N, D), jnp.bfloat16)],
        compiler_params=pltpu.CompilerParams(
            dimension_semantics=("arbitrary",)),
    )(norm_adj, user_embedding, item_embedding)

    def _u_map(t):
        l, i = t // NBLK2, t % NBLK2
        return (jnp.where(l == 1, jnp.minimum(i, UBLK2 - 1), 0), 0)

    def _v_map(t):
        l, i = t // NBLK2, t % NBLK2
        return (jnp.where(l == 1, jnp.maximum(i - UBLK2, 0), 0), 0)

    u_out, v_out = pl.pallas_call(
        _pass2_kernel,
        grid=(2 * NBLK2,),
        in_specs=[
            pl.BlockSpec((BM2, N), lambda t: (t % NBLK2, 0)),
            pl.BlockSpec((N, D), lambda t: (0, 0)),
            pl.BlockSpec((N_USER, D), lambda t: (0, 0)),
            pl.BlockSpec((N - N_USER, D), lambda t: (0, 0)),
        ],
        out_specs=[
            pl.BlockSpec((BM2, D), _u_map),
            pl.BlockSpec((BM2, D), _v_map),
        ],
        out_shape=[
            jax.ShapeDtypeStruct((N_USER, D), jnp.float32),
            jax.ShapeDtypeStruct((N - N_USER, D), jnp.float32),
        ],
        scratch_shapes=[
            pltpu.VMEM((N, D), jnp.bfloat16),
        ],
        compiler_params=pltpu.CompilerParams(
            dimension_semantics=("arbitrary",)),
    )(a16, r1, user_embedding, item_embedding)
    return (u_out, v_out)
